# baseline (speedup 1.0000x reference)
import jax
import jax.numpy as jnp
from jax.experimental import pallas as pl
from jax.experimental.pallas import tpu as pltpu

_BN_EPS = 1e-5
_NORM_EPS = 1e-12
_VMEM_LIMIT = 48 * 1024 * 1024

_Q = 8


def _conv_gap_body(g_ref, w_ref, o_ref, *, img_tile, h_img, w_img, f_dim):
    wq = w_img // _Q
    hwq = h_img * wq
    w = w_ref[...]
    inv = 1.0 / (h_img * w_img)
    for i in range(img_tile):
        p = jnp.concatenate(
            [g_ref[i, 0:hwq, :],
             g_ref[i, wq:wq + hwq, 0:30],
             g_ref[i, 2 * wq:2 * wq + hwq, 0:30]],
            axis=1)
        y = jnp.dot(p, w, preferred_element_type=jnp.float32)
        y = jnp.maximum(y, 0.0)
        s = jnp.sum(y, axis=0, keepdims=True)
        s = sum(s[:, k * f_dim:(k + 1) * f_dim] for k in range(_Q))
        o_ref[pl.ds(i, 1), :] = (s * inv).astype(o_ref.dtype)


def _conv_gap(g, w_ext, *, h_img, w_img, f_dim, img_tile=128):
    BB, rows, L = g.shape
    body = lambda gr, wr, o: _conv_gap_body(
        gr, wr, o, img_tile=img_tile, h_img=h_img, w_img=w_img, f_dim=f_dim)
    return pl.pallas_call(
        body,
        out_shape=jax.ShapeDtypeStruct((BB, f_dim), jnp.bfloat16),
        grid=(BB // img_tile,),
        in_specs=[
            pl.BlockSpec((img_tile, rows, L), lambda b: (b, 0, 0)),
            pl.BlockSpec(w_ext.shape, lambda b: (0, 0)),
        ],
        out_specs=pl.BlockSpec((img_tile, f_dim), lambda b: (b, 0)),
        compiler_params=pltpu.CompilerParams(
            dimension_semantics=("parallel",),
            vmem_limit_bytes=_VMEM_LIMIT),
    )(g, w_ext)


def _heads_body(f1, f2,
                ow1, ob1, og, obt, ow2, ob2,
                pw1, pb1, pg, pbt, pw2, pb2,
                tw1, tb1, tg, tbt, tw2, tb2,
                o_ref):
    def head(x, w1, b1, g, bt, w2, b2):
        pre = jnp.dot(x, w1[...], preferred_element_type=jnp.float32) + b1[...]
        mu = jnp.mean(pre, axis=0, keepdims=True)
        d = pre - mu
        var = jnp.mean(d * d, axis=0, keepdims=True)
        act = jnp.maximum(d * jax.lax.rsqrt(var + _BN_EPS) * g[...] + bt[...],
                          0.0)
        return jnp.dot(act.astype(w2.dtype), w2[...],
                       preferred_element_type=jnp.float32) + b2[...]

    z1 = head(f1[...], ow1, ob1, og, obt, ow2, ob2)
    z2 = head(f2[...], tw1, tb1, tg, tbt, tw2, tb2)
    q = head(z1.astype(pw1.dtype), pw1, pb1, pg, pbt, pw2, pb2)
    inv1 = jax.lax.rsqrt(jnp.maximum(
        jnp.sum(q * q, axis=-1, keepdims=True), _NORM_EPS * _NORM_EPS))
    inv2 = jax.lax.rsqrt(jnp.maximum(
        jnp.sum(z2 * z2, axis=-1, keepdims=True), _NORM_EPS * _NORM_EPS))
    sim = jnp.sum((q * inv1) * (z2 * inv2), axis=-1)
    o_ref[0] = 2.0 - 2.0 * (jnp.sum(sim) / q.shape[0])


def _heads_loss(f1, f2, args):
    def full(a):
        nd = a.ndim
        return pl.BlockSpec(a.shape, lambda _nd=nd: (0,) * _nd)

    ops = [f1, f2] + list(args)
    out = pl.pallas_call(
        _heads_body,
        out_shape=jax.ShapeDtypeStruct((1,), jnp.float32),
        grid=(),
        in_specs=[full(a) for a in ops],
        out_specs=pl.BlockSpec(memory_space=pltpu.MemorySpace.SMEM),
        compiler_params=pltpu.CompilerParams(
            vmem_limit_bytes=_VMEM_LIMIT),
    )(*ops)
    return out[0]


def _pack_rows(x_nchw):
    x = x_nchw.astype(jnp.bfloat16)
    B, C, H, W = x.shape
    wq = W // _Q
    xp = jnp.pad(x, ((0, 0), (0, 0), (1, 1), (1, 1)))
    per_w16 = []
    for w16 in range(wq):
        pieces = [xp[:, c, :, _Q * w16:_Q * w16 + _Q + 2] for c in range(C)]
        pieces.append(jnp.ones((B, H + 2, 2), jnp.bfloat16))
        per_w16.append(jnp.concatenate(pieces, axis=-1))
    g = jnp.stack(per_w16, axis=2)
    return g.reshape(B, (H + 2) * wq, C * (_Q + 2) + 2)


def _pack_weights(conv_w, conv_b, f_dim):
    W = conv_w.reshape(3, 3, 3, f_dim)
    zero = jnp.zeros((f_dim,), conv_w.dtype)
    b_hi = conv_b.astype(jnp.bfloat16)
    b_lo = (conv_b - b_hi.astype(jnp.float32)).astype(jnp.bfloat16)
    blocks = []
    for dh in range(3):
        rows = []
        for c in range(3):
            for col in range(_Q + 2):
                lanes = [W[dh, col - wi, c] if 0 <= col - wi < 3 else zero
                         for wi in range(_Q)]
                rows.append(jnp.concatenate(lanes, axis=0))
        blk = jnp.stack(rows, axis=0)
        if dh == 0:
            bias = jnp.concatenate([jnp.tile(b_hi, (1, _Q)),
                                    jnp.tile(b_lo, (1, _Q))], axis=0)
            blk = jnp.concatenate([blk, bias], axis=0)
        blocks.append(blk)
    return jnp.concatenate(blocks, axis=0)


def kernel(x1, x2, conv_w, conv_b,
           on_w1, on_b1, on_gamma, on_beta, on_w2, on_b2,
           pr_w1, pr_b1, pr_gamma, pr_beta, pr_w2, pr_b2,
           tg_w1, tg_b1, tg_gamma, tg_beta, tg_w2, tg_b2):
    B = x1.shape[0]
    H, W = x1.shape[2], x1.shape[3]
    F = conv_w.shape[1]
    g = _pack_rows(jnp.concatenate(
        [x1.astype(jnp.bfloat16), x2.astype(jnp.bfloat16)], axis=0))
    w_ext = _pack_weights(conv_w, conv_b, F)

    f = _conv_gap(g, w_ext, h_img=H, w_img=W, f_dim=F)
    f1, f2 = f[:B], f[B:]
    return _heads_loss(f1, f2, [
        on_w1, on_b1, on_gamma, on_beta, on_w2, on_b2,
        pr_w1, pr_b1, pr_gamma, pr_beta, pr_w2, pr_b2,
        tg_w1, tg_b1, tg_gamma, tg_beta, tg_w2, tg_b2])


# speedup vs baseline: 1.2867x; 1.2867x over previous
import jax
import jax.numpy as jnp
from jax.experimental import pallas as pl
from jax.experimental.pallas import tpu as pltpu

_BN_EPS = 1e-5
_NORM_EPS = 1e-12
_VMEM_LIMIT = 48 * 1024 * 1024

_Q = 8


def _conv_gap_body(g_ref, w_ref, o_ref, *, img_tile, h_img, w_img, f_dim):
    wq = w_img // _Q
    hwq = h_img * wq
    w = w_ref[...]
    inv = 1.0 / (h_img * w_img)
    for i in range(img_tile):
        p = jnp.concatenate(
            [g_ref[i, 0:hwq, :],
             g_ref[i, wq:wq + hwq, 0:30],
             g_ref[i, 2 * wq:2 * wq + hwq, 0:30]],
            axis=1)
        y = jnp.dot(p, w, preferred_element_type=jnp.float32)
        y = jnp.maximum(y, 0.0)
        s = jnp.sum(y, axis=0, keepdims=True)
        s = sum(s[:, k * f_dim:(k + 1) * f_dim] for k in range(_Q))
        o_ref[pl.ds(i, 1), :] = (s * inv).astype(o_ref.dtype)


def _conv_gap(g, w_ext, *, h_img, w_img, f_dim, img_tile=64):
    BB, rows, L = g.shape
    img_tile = min(img_tile, BB)
    body = lambda gr, wr, o: _conv_gap_body(
        gr, wr, o, img_tile=img_tile, h_img=h_img, w_img=w_img, f_dim=f_dim)
    return pl.pallas_call(
        body,
        out_shape=jax.ShapeDtypeStruct((BB, f_dim), jnp.bfloat16),
        grid=(BB // img_tile,),
        in_specs=[
            pl.BlockSpec((img_tile, rows, L), lambda b: (b, 0, 0)),
            pl.BlockSpec(w_ext.shape, lambda b: (0, 0)),
        ],
        out_specs=pl.BlockSpec((img_tile, f_dim), lambda b: (b, 0)),
        compiler_params=pltpu.CompilerParams(
            dimension_semantics=("parallel",),
            vmem_limit_bytes=_VMEM_LIMIT),
    )(g, w_ext)


def _heads_body(f1, f2,
                ow1, ob1, og, obt, ow2, ob2,
                pw1, pb1, pg, pbt, pw2, pb2,
                tw1, tb1, tg, tbt, tw2, tb2,
                o_ref):
    def head(x, w1, b1, g, bt, w2, b2):
        pre = jnp.dot(x, w1[...], preferred_element_type=jnp.float32) + b1[...]
        mu = jnp.mean(pre, axis=0, keepdims=True)
        d = pre - mu
        var = jnp.mean(d * d, axis=0, keepdims=True)
        act = jnp.maximum(d * jax.lax.rsqrt(var + _BN_EPS) * g[...] + bt[...],
                          0.0)
        return jnp.dot(act.astype(w2.dtype), w2[...],
                       preferred_element_type=jnp.float32) + b2[...]

    z1 = head(f1[...], ow1, ob1, og, obt, ow2, ob2)
    z2 = head(f2[...], tw1, tb1, tg, tbt, tw2, tb2)
    q = head(z1.astype(pw1.dtype), pw1, pb1, pg, pbt, pw2, pb2)
    inv1 = jax.lax.rsqrt(jnp.maximum(
        jnp.sum(q * q, axis=-1, keepdims=True), _NORM_EPS * _NORM_EPS))
    inv2 = jax.lax.rsqrt(jnp.maximum(
        jnp.sum(z2 * z2, axis=-1, keepdims=True), _NORM_EPS * _NORM_EPS))
    sim = jnp.sum((q * inv1) * (z2 * inv2), axis=-1)
    o_ref[0] = 2.0 - 2.0 * (jnp.sum(sim) / q.shape[0])


def _heads_loss(f1, f2, args):
    def full(a):
        nd = a.ndim
        return pl.BlockSpec(a.shape, lambda _nd=nd: (0,) * _nd)

    ops = [f1, f2] + list(args)
    out = pl.pallas_call(
        _heads_body,
        out_shape=jax.ShapeDtypeStruct((1,), jnp.float32),
        grid=(),
        in_specs=[full(a) for a in ops],
        out_specs=pl.BlockSpec(memory_space=pltpu.MemorySpace.SMEM),
        compiler_params=pltpu.CompilerParams(
            vmem_limit_bytes=_VMEM_LIMIT),
    )(*ops)
    return out[0]


def _pack_rows(x_nchw):
    x = x_nchw.astype(jnp.bfloat16)
    B, C, H, W = x.shape
    wq = W // _Q
    xp = jnp.pad(x, ((0, 0), (0, 0), (1, 1), (1, 1)))
    per_wq = []
    for w8 in range(wq):
        pieces = [xp[:, c, :, _Q * w8:_Q * w8 + _Q + 2] for c in range(C)]
        pieces.append(jnp.ones((B, H + 2, 2), jnp.bfloat16))
        per_wq.append(jnp.concatenate(pieces, axis=-1))
    g = jnp.stack(per_wq, axis=2)
    return g.reshape(B, (H + 2) * wq, C * (_Q + 2) + 2)


def _pack_weights(conv_w, conv_b, f_dim):
    W = conv_w.reshape(3, 3, 3, f_dim)
    zero = jnp.zeros((f_dim,), conv_w.dtype)
    b_hi = conv_b.astype(jnp.bfloat16)
    b_lo = (conv_b - b_hi.astype(jnp.float32)).astype(jnp.bfloat16)
    blocks = []
    for dh in range(3):
        rows = []
        for c in range(3):
            for col in range(_Q + 2):
                lanes = [W[dh, col - wi, c] if 0 <= col - wi < 3 else zero
                         for wi in range(_Q)]
                rows.append(jnp.concatenate(lanes, axis=0))
        blk = jnp.stack(rows, axis=0)
        if dh == 0:
            bias = jnp.concatenate([jnp.tile(b_hi, (1, _Q)),
                                    jnp.tile(b_lo, (1, _Q))], axis=0)
            blk = jnp.concatenate([blk, bias], axis=0)
        blocks.append(blk)
    return jnp.concatenate(blocks, axis=0)


def kernel(x1, x2, conv_w, conv_b,
           on_w1, on_b1, on_gamma, on_beta, on_w2, on_b2,
           pr_w1, pr_b1, pr_gamma, pr_beta, pr_w2, pr_b2,
           tg_w1, tg_b1, tg_gamma, tg_beta, tg_w2, tg_b2):
    B = x1.shape[0]
    H, W = x1.shape[2], x1.shape[3]
    F = conv_w.shape[1]
    g = _pack_rows(jnp.concatenate(
        [x1.astype(jnp.bfloat16), x2.astype(jnp.bfloat16)], axis=0))
    w_ext = _pack_weights(conv_w, conv_b, F)

    f = _conv_gap(g, w_ext, h_img=H, w_img=W, f_dim=F)
    f1, f2 = f[:B], f[B:]
    return _heads_loss(f1, f2, [
        on_w1, on_b1, on_gamma, on_beta, on_w2, on_b2,
        pr_w1, pr_b1, pr_gamma, pr_beta, pr_w2, pr_b2,
        tg_w1, tg_b1, tg_gamma, tg_beta, tg_w2, tg_b2])
